# baseline (speedup 1.0000x reference)
import functools
import math

import jax
import jax.numpy as jnp
from jax import lax
from jax.experimental import pallas as pl
from jax.experimental.pallas import tpu as pltpu
from jax.experimental.pallas import tpu_sc as plsc

F32 = jnp.float32
I32 = jnp.int32
INF = float("inf")
IBIG = 2**31 - 1
NEG = -1e30


def _iota(shape, dim):
    return lax.broadcasted_iota(I32, shape, dim)


def _fps_body(n_sample, n_valid, W, pos_ref, out_ref):
    px = pos_ref[0, 0]
    py = pos_ref[0, 1]
    pz = pos_ref[0, 2]
    idxmat = _iota((8, W), 0) * W + _iota((8, W), 1)
    valid = idxmat < n_valid
    lane = _iota((1, 128), 1)

    def extract(nxt):
        m = idxmat == nxt
        lx = jnp.sum(jnp.where(m, px, 0.0), axis=(0, 1), keepdims=True)
        ly = jnp.sum(jnp.where(m, py, 0.0), axis=(0, 1), keepdims=True)
        lz = jnp.sum(jnp.where(m, pz, 0.0), axis=(0, 1), keepdims=True)
        return lx, ly, lz

    def store_row(i, lx, ly, lz):
        row = jnp.where(lane == 0, lx,
                        jnp.where(lane == 1, ly,
                                  jnp.where(lane == 2, lz, 0.0)))
        out_ref[0, pl.ds(i, 1), :] = row

    lx, ly, lz = extract(jnp.zeros((1, 1), I32))
    store_row(0, lx, ly, lz)
    dists0 = jnp.where(valid, INF, -INF)

    def body(i, carry):
        dists, lx, ly, lz = carry
        dx = px - lx
        dy = py - ly
        dz = pz - lz
        d = (dx * dx + dy * dy) + dz * dz
        dists = jnp.minimum(dists, d)
        m = jnp.max(dists, axis=(0, 1), keepdims=True)
        nxt = jnp.min(jnp.where(dists == m, idxmat, IBIG),
                      axis=(0, 1), keepdims=True)
        lx, ly, lz = extract(nxt)
        store_row(i, lx, ly, lz)
        return dists, lx, ly, lz

    lax.fori_loop(1, n_sample, body, (dists0, lx, ly, lz))
    n_pad = out_ref.shape[1]
    if n_pad > n_sample:
        out_ref[0, n_sample:, :] = jnp.zeros((n_pad - n_sample, 128), F32)


def _fps(planes, n_sample, n_pad, n_valid):
    B, _, _, W = planes.shape
    return pl.pallas_call(
        functools.partial(_fps_body, n_sample, n_valid, W),
        grid=(B,),
        in_specs=[pl.BlockSpec((1, 3, 8, W), lambda b: (b, 0, 0, 0))],
        out_specs=pl.BlockSpec((1, n_pad, 128), lambda b: (b, 0, 0)),
        out_shape=jax.ShapeDtypeStruct((B, n_pad, 128), F32),
        compiler_params=pltpu.CompilerParams(
            dimension_semantics=("parallel",)),
        interpret=False,
    )(planes)


def _select_body(k, n_pts, r2, weighted, Np, Rc,
                 c_ref, p_ref, nbr_ref, aux_ref, d2m_ref):
    lane = _iota((1, Np), 1)
    col = _iota((Rc, k), 1)
    cx = c_ref[0][:, 0:1]
    cy = c_ref[0][:, 1:2]
    cz = c_ref[0][:, 2:3]
    px = p_ref[0, 0:1, :]
    py = p_ref[0, 1:2, :]
    pz = p_ref[0, 2:3, :]
    dx = cx - px
    dy = cy - py
    dz = cz - pz
    d2 = (dx * dx + dy * dy) + dz * dz
    ok = lane < n_pts
    if r2 is not None:
        ok = ok & (d2 <= r2)
    d2m_ref[...] = jnp.where(ok, d2, INF)

    nbr0 = jnp.zeros((Rc, k), I32)
    vld0 = jnp.zeros((Rc, k), I32)
    val0 = jnp.zeros((Rc, k), F32)

    def round_fn(r, nbr, vld, vals):
        d2m = d2m_ref[...]
        m = jnp.min(d2m, axis=1, keepdims=True)
        sel = jnp.min(jnp.where(d2m == m, lane, IBIG), axis=1, keepdims=True)
        has = m < INF
        upd = (col == r) & has
        nbr = jnp.where(upd, sel, nbr)
        vld = jnp.where(upd, 1, vld)
        vals = jnp.where(upd, m, vals)
        d2m_ref[...] = jnp.where(lane == sel, INF, d2m)
        return nbr, vld, vals, jnp.min(m) < INF

    if k > 4:
        def wbody(carry):
            r, nbr, vld, vals, _ = carry
            nbr, vld, vals, cont = round_fn(r, nbr, vld, vals)
            return r + 1, nbr, vld, vals, cont

        _, nbr, vld, vals, _ = lax.while_loop(
            lambda c: (c[0] < k) & c[4], wbody,
            (jnp.int32(0), nbr0, vld0, val0, True))
    else:
        nbr, vld, vals = nbr0, vld0, val0
        for r in range(k):
            nbr, vld, vals, _ = round_fn(jnp.int32(r), nbr, vld, vals)

    nbr_ref[0] = nbr
    if weighted:
        w = 1.0 / jnp.maximum(vals, 1e-16)
        aux_ref[0] = w / jnp.sum(w, axis=-1, keepdims=True)
    else:
        aux_ref[0] = vld


def _select(centers, pointsT, n_pts, k, r2, weighted, Rc=128):
    B, n_pad, _ = centers.shape
    Np = pointsT.shape[2]
    aux_dt = F32 if weighted else I32
    return pl.pallas_call(
        functools.partial(_select_body, k, n_pts, r2, weighted, Np, Rc),
        grid=(B, n_pad // Rc),
        in_specs=[
            pl.BlockSpec((1, Rc, 128), lambda b, i: (b, i, 0)),
            pl.BlockSpec((1, 8, Np), lambda b, i: (b, 0, 0)),
        ],
        out_specs=[
            pl.BlockSpec((1, Rc, k), lambda b, i: (b, i, 0)),
            pl.BlockSpec((1, Rc, k), lambda b, i: (b, i, 0)),
        ],
        out_shape=[
            jax.ShapeDtypeStruct((B, n_pad, k), I32),
            jax.ShapeDtypeStruct((B, n_pad, k), aux_dt),
        ],
        scratch_shapes=[pltpu.VMEM((Rc, Np), F32)],
        compiler_params=pltpu.CompilerParams(
            dimension_semantics=("parallel", "parallel")),
        interpret=False,
    )(centers, pointsT)


def _mm_body(has_bias, refs):
    if has_bias:
        x_ref, w_ref, b_ref, o_ref = refs
    else:
        x_ref, w_ref, o_ref = refs
    o = jnp.dot(x_ref[0], w_ref[...], preferred_element_type=F32)
    if has_bias:
        o = o + b_ref[...]
    o_ref[0] = o


def _mm(x, W, b=None, BM=512):
    B, M, K = x.shape
    H = W.shape[1]
    has_bias = b is not None
    ins = [x, W] + ([b.reshape(1, H)] if has_bias else [])
    in_specs = [
        pl.BlockSpec((1, BM, K), lambda bb, i: (bb, i, 0)),
        pl.BlockSpec((K, H), lambda bb, i: (0, 0)),
    ] + ([pl.BlockSpec((1, H), lambda bb, i: (0, 0))] if has_bias else [])
    return pl.pallas_call(
        lambda *refs: _mm_body(has_bias, refs),
        grid=(B, M // BM),
        in_specs=in_specs,
        out_specs=pl.BlockSpec((1, BM, H), lambda bb, i: (bb, i, 0)),
        out_shape=jax.ShapeDtypeStruct((B, M, H), F32),
        compiler_params=pltpu.CompilerParams(
            dimension_semantics=("parallel", "parallel")),
        interpret=False,
    )(*ins)


def _gather_plan(H):
    if H <= 128:
        return 128, 4
    if H <= 256:
        return 128, 2
    return 64, 2


def _sc_gather(table, idx):
    V, H = table.shape
    Bp = idx.shape[0]
    CH, K = _gather_plan(H)
    info = plsc.get_sparse_core_info()
    NC, NS = info.num_cores, info.num_subcores
    NW = NC * NS
    rows_per_w = (Bp // CH) // NW
    groups = rows_per_w // K
    mesh = plsc.VectorSubcoreMesh(core_axis_name="c", subcore_axis_name="s")

    @functools.partial(
        pl.kernel, mesh=mesh,
        out_type=jax.ShapeDtypeStruct((Bp // CH, CH, H), F32),
        scratch_types=[
            pltpu.VMEM((K, CH), I32),
            pltpu.VMEM((K, CH, H), F32),
            pltpu.SemaphoreType.DMA,
        ],
    )
    def gk(table_hbm, idx_hbm, out_hbm, idx_v, rows_v, sem):
        wid = lax.axis_index("s") * NC + lax.axis_index("c")

        def body(g, carry):
            row0 = wid * rows_per_w + g * K
            pltpu.sync_copy(idx_hbm.at[pl.ds(row0, K)], idx_v)
            handles = [
                pltpu.async_copy(table_hbm.at[idx_v.at[j]], rows_v.at[j], sem)
                for j in range(K)
            ]
            for h in handles:
                h.wait()
            pltpu.sync_copy(rows_v, out_hbm.at[pl.ds(row0, K)])
            return carry

        lax.fori_loop(0, groups, body, jnp.int32(0))

    return gk(table, idx.reshape(Bp // CH, CH)).reshape(Bp, H)


def _gather_rows(table2, flat_idx):
    Bn = flat_idx.shape[0]
    CH, K = _gather_plan(table2.shape[1])
    grain = 32 * CH * K
    Bp = ((Bn + grain - 1) // grain) * grain
    if Bp != Bn:
        flat_idx = jnp.pad(flat_idx, (0, Bp - Bn))
    out = _sc_gather(table2, flat_idx)
    return out[:Bn]


def _pair_mlp_body(Rc, H1, n_layers, refs):
    g_ref, c_ref, wp_ref, vld_ref = refs[:4]
    wbs = refs[4:4 + 2 * n_layers]
    o_ref = refs[4 + 2 * n_layers]
    cx = c_ref[0][:, 0:1]
    cy = c_ref[0][:, 1:2]
    cz = c_ref[0][:, 2:3]
    T = cx * wp_ref[0:1, :] + cy * wp_ref[1:2, :] + cz * wp_ref[2:3, :]
    g = g_ref[0]
    if g.shape[1] > H1:
        g = g[:, :H1]
    g3 = g.reshape(Rc, 32, H1)
    h3 = jnp.maximum(g3 - T[:, None, :], 0.0)
    h = h3.reshape(Rc * 32, H1)
    for li in range(n_layers):
        W = wbs[2 * li][...]
        b = wbs[2 * li + 1][...]
        h = jnp.maximum(jnp.dot(h, W, preferred_element_type=F32) + b, 0.0)
    HL = h.shape[1]
    h3 = h.reshape(Rc, 32, HL)
    vld3 = vld_ref[0][:, :, None] > 0
    o_ref[0] = jnp.max(jnp.where(vld3, h3, NEG), axis=1)


def _pair_mlp(G, centers, Wp, vld, layers, Rc=128):
    B, n_pad, _ = centers.shape
    Hp = G.shape[2]
    H1 = Wp.shape[1]
    HL = layers[-1][0].shape[1] if layers else H1
    ins = [G, centers, Wp, vld]
    in_specs = [
        pl.BlockSpec((1, Rc * 32, Hp), lambda b, i: (b, i, 0)),
        pl.BlockSpec((1, Rc, 128), lambda b, i: (b, i, 0)),
        pl.BlockSpec(Wp.shape, lambda b, i: (0, 0)),
        pl.BlockSpec((1, Rc, 32), lambda b, i: (b, i, 0)),
    ]
    for (W, bb) in layers:
        ins += [W, bb.reshape(1, -1)]
        in_specs += [pl.BlockSpec(W.shape, lambda b, i: (0, 0)),
                     pl.BlockSpec((1, bb.shape[0]), lambda b, i: (0, 0))]
    return pl.pallas_call(
        lambda *refs: _pair_mlp_body(Rc, H1, len(layers), refs),
        grid=(B, n_pad // Rc),
        in_specs=in_specs,
        out_specs=pl.BlockSpec((1, Rc, HL), lambda b, i: (b, i, 0)),
        out_shape=jax.ShapeDtypeStruct((B, n_pad, HL), F32),
        compiler_params=pltpu.CompilerParams(
            dimension_semantics=("parallel", "parallel")),
        interpret=False,
    )(*ins)


def _fp_mlp(G0, G1, G2, wts, skip, pos_rows, w1b, w1c, b1, layers,
            resid=None, Rc=128):
    B, n_pad, H1 = G0.shape
    HL = layers[-1][0].shape[1] if layers else H1

    def body(*refs):
        g0, g1, g2, w_ref, sk_ref, pr_ref, w1b_r, w1c_r, b1_r = refs[:9]
        p = 9
        h_w = []
        for (W, bb, do_relu) in layers:
            Wr = refs[p]
            p += 1
            br = None
            if bb is not None:
                br = refs[p]
                p += 1
            h_w.append((Wr, br, do_relu))
        resid_ref = refs[p] if resid is not None else None
        if resid is not None:
            p += 1
        o_ref = refs[p]
        w = w_ref[0]
        interp = (w[:, 0:1] * g0[0] + w[:, 1:2] * g1[0]) + w[:, 2:3] * g2[0]
        sk = jnp.dot(sk_ref[0], w1b_r[...], preferred_element_type=F32)
        px = pr_ref[0][:, 0:1]
        py = pr_ref[0][:, 1:2]
        pz = pr_ref[0][:, 2:3]
        pp = px * w1c_r[0:1, :] + py * w1c_r[1:2, :] + pz * w1c_r[2:3, :]
        h = jnp.maximum(interp + sk + pp + b1_r[...], 0.0)
        for (Wr, br, do_relu) in h_w:
            h = jnp.dot(h, Wr[...], preferred_element_type=F32)
            if br is not None:
                h = h + br[...]
            if do_relu:
                h = jnp.maximum(h, 0.0)
        if resid_ref is not None:
            h = h + resid_ref[0]
        o_ref[0] = h

    Cs = skip.shape[2]
    ins = [G0, G1, G2, wts, skip, pos_rows, w1b, w1c, b1.reshape(1, -1)]
    in_specs = [
        pl.BlockSpec((1, Rc, H1), lambda b, i: (b, i, 0)),
        pl.BlockSpec((1, Rc, H1), lambda b, i: (b, i, 0)),
        pl.BlockSpec((1, Rc, H1), lambda b, i: (b, i, 0)),
        pl.BlockSpec((1, Rc, 3), lambda b, i: (b, i, 0)),
        pl.BlockSpec((1, Rc, Cs), lambda b, i: (b, i, 0)),
        pl.BlockSpec((1, Rc, 128), lambda b, i: (b, i, 0)),
        pl.BlockSpec(w1b.shape, lambda b, i: (0, 0)),
        pl.BlockSpec(w1c.shape, lambda b, i: (0, 0)),
        pl.BlockSpec((1, b1.shape[0]), lambda b, i: (0, 0)),
    ]
    for (W, bb, _r) in layers:
        ins.append(W)
        in_specs.append(pl.BlockSpec(W.shape, lambda b, i: (0, 0)))
        if bb is not None:
            ins.append(bb.reshape(1, -1))
            in_specs.append(pl.BlockSpec((1, bb.shape[0]),
                                         lambda b, i: (0, 0)))
    if resid is not None:
        ins.append(resid)
        in_specs.append(pl.BlockSpec((1, Rc, resid.shape[2]),
                                     lambda b, i: (b, i, 0)))
    return pl.pallas_call(
        body,
        grid=(B, n_pad // Rc),
        in_specs=in_specs,
        out_specs=pl.BlockSpec((1, Rc, HL), lambda b, i: (b, i, 0)),
        out_shape=jax.ShapeDtypeStruct((B, n_pad, HL), F32),
        compiler_params=pltpu.CompilerParams(
            dimension_semantics=("parallel", "parallel")),
        interpret=False,
    )(*ins)


def _rows_to_planes(rows, n_pad):
    t = rows[:, :, :3].transpose(0, 2, 1)
    return t.reshape(2, 3, 8, n_pad // 8)


def _rows_to_pT(rows):
    t = rows[:, :, :3].transpose(0, 2, 1)
    return jnp.pad(t, ((0, 0), (0, 5), (0, 0)))


def _sa_stage(prm, x_prev, pos_rows, planes, pT, n_prev, np_prev,
              n_s, np_s, radius):
    W1, b1 = prm[0]
    Cx = x_prev.shape[2]
    H1 = W1.shape[1]
    Hp = -(-H1 // 128) * 128
    centers = _fps(planes, n_s, np_s, n_prev)
    nbr, vld = _select(centers, pT, n_prev, 32, radius * radius, False)
    xp = jnp.concatenate([x_prev, pos_rows[:, :, :3]], axis=-1)
    W1p = jnp.pad(W1, ((0, 0), (0, Hp - H1)))
    b1p = jnp.pad(b1, (0, Hp - H1))
    A = _mm(xp, W1p, b1p)
    flat_idx = (nbr + jnp.arange(2, dtype=I32)[:, None, None] * np_prev)
    G = _gather_rows(A.reshape(2 * np_prev, -1), flat_idx.reshape(-1))
    G = G.reshape(2, np_s * 32, -1)
    x_s = _pair_mlp(G, centers, W1[Cx:], vld, prm[1:])
    return x_s, centers


def _fp_stage(prm, x_c, pos_c_rows, x_skip, pos_f_rows, n_c, np_c,
              n_f, np_f, extra_fc=None, resid=None):
    W1, b1 = prm[0]
    Cc = x_c.shape[2]
    Cs = x_skip.shape[2]
    pT_c = _rows_to_pT(pos_c_rows)
    nbr, w = _select(pos_f_rows, pT_c, n_c, 3, None, True)
    Bmat = _mm(x_c, W1[:Cc])
    fi = (nbr + jnp.arange(2, dtype=I32)[:, None, None] * np_c)
    fi = fi.transpose(2, 0, 1).reshape(-1)
    G = _gather_rows(Bmat.reshape(2 * np_c, -1), fi)
    G = G.reshape(3, 2, np_f, -1)
    layers = [(W, b, True) for (W, b) in prm[1:]]
    if extra_fc is not None:
        layers.append((extra_fc[0], None, True))
        for Wf in extra_fc[1:]:
            layers.append((Wf, None, False))
    return _fp_mlp(G[0], G[1], G[2], w, x_skip, pos_f_rows,
                   W1[Cc:Cc + Cs], W1[Cc + Cs:], b1, layers, resid=resid)


def kernel(data, params):
    n = [4096, 2868, 2008, 1406, 985]
    npad = [4096, 3072, 2048, 1536, 1024]
    radii = [0.025, 0.05, 0.1, 0.2]

    pos0_rows3 = data.transpose(0, 2, 1)
    pos0_rows = jnp.pad(pos0_rows3, ((0, 0), (0, 0), (0, 125)))
    planes0 = data.reshape(2, 3, 8, 512)
    pT0 = jnp.pad(data, ((0, 0), (0, 5), (0, 0)))

    xs = [pos0_rows3]
    ps = [pos0_rows]
    planes, pT = planes0, pT0
    for s in range(4):
        prm = params['sa' + str(s + 1)]
        x_s, centers = _sa_stage(
            prm, xs[s], ps[s], planes, pT,
            n[s], npad[s], n[s + 1], npad[s + 1], radii[s])
        xs.append(x_s)
        ps.append(centers)
        if s < 3:
            planes = _rows_to_planes(centers, npad[s + 1])
            pT = _rows_to_pT(centers)

    f4 = _fp_stage(params['fp4'], xs[4], ps[4], xs[3], ps[3],
                   n[4], npad[4], n[3], npad[3])
    f3 = _fp_stage(params['fp3'], f4, ps[3], xs[2], ps[2],
                   n[3], npad[3], n[2], npad[2])
    f2 = _fp_stage(params['fp2'], f3, ps[2], xs[1], ps[1],
                   n[2], npad[2], n[1], npad[1])
    out = _fp_stage(params['fp1'], f2, ps[1], pos0_rows3, ps[0],
                    n[1], npad[1], n[0], npad[0],
                    extra_fc=params['fc'], resid=pos0_rows3)
    return out.transpose(0, 2, 1)


# speedup vs baseline: 1.6385x; 1.6385x over previous
import functools
import math

import jax
import jax.numpy as jnp
from jax import lax
from jax.experimental import pallas as pl
from jax.experimental.pallas import tpu as pltpu

F32 = jnp.float32
I32 = jnp.int32
INF = float("inf")
IBIG = 2**31 - 1
NEG = -1e30


def _iota(shape, dim):
    return lax.broadcasted_iota(I32, shape, dim)


def _fps_body(n_sample, n_valid, W, pos_ref, out_ref):
    px = pos_ref[0, 0]
    py = pos_ref[0, 1]
    pz = pos_ref[0, 2]
    idxmat = _iota((8, W), 0) * W + _iota((8, W), 1)
    valid = idxmat < n_valid
    lane = _iota((1, 128), 1)

    def extract(nxt):
        m = idxmat == nxt
        lx = jnp.sum(jnp.where(m, px, 0.0), axis=(0, 1), keepdims=True)
        ly = jnp.sum(jnp.where(m, py, 0.0), axis=(0, 1), keepdims=True)
        lz = jnp.sum(jnp.where(m, pz, 0.0), axis=(0, 1), keepdims=True)
        return lx, ly, lz

    def store_row(i, lx, ly, lz):
        row = jnp.where(lane == 0, lx,
                        jnp.where(lane == 1, ly,
                                  jnp.where(lane == 2, lz, 0.0)))
        out_ref[0, pl.ds(i, 1), :] = row

    lx, ly, lz = extract(jnp.zeros((1, 1), I32))
    store_row(0, lx, ly, lz)
    dists0 = jnp.where(valid, INF, -INF)

    def body(i, carry):
        dists, lx, ly, lz = carry
        dx = px - lx
        dy = py - ly
        dz = pz - lz
        d = (dx * dx + dy * dy) + dz * dz
        dists = jnp.minimum(dists, d)
        m = jnp.max(dists, axis=(0, 1), keepdims=True)
        nxt = jnp.min(jnp.where(dists == m, idxmat, IBIG),
                      axis=(0, 1), keepdims=True)
        lx, ly, lz = extract(nxt)
        store_row(i, lx, ly, lz)
        return dists, lx, ly, lz

    lax.fori_loop(1, n_sample, body, (dists0, lx, ly, lz))
    n_pad = out_ref.shape[1]
    if n_pad > n_sample:
        out_ref[0, n_sample:, :] = jnp.zeros((n_pad - n_sample, 128), F32)


def _fps(planes, n_sample, n_pad, n_valid):
    B, _, _, W = planes.shape
    return pl.pallas_call(
        functools.partial(_fps_body, n_sample, n_valid, W),
        grid=(B,),
        in_specs=[pl.BlockSpec((1, 3, 8, W), lambda b: (b, 0, 0, 0))],
        out_specs=pl.BlockSpec((1, n_pad, 128), lambda b: (b, 0, 0)),
        out_shape=jax.ShapeDtypeStruct((B, n_pad, 128), F32),
        compiler_params=pltpu.CompilerParams(
            dimension_semantics=("parallel",)),
        interpret=False,
    )(planes)


def _select_body(k, n_pts, r2, weighted, Np, Rc,
                 c_ref, p_ref, nbr_ref, aux_ref, d2m_ref):
    lane = _iota((1, Np), 1)
    col = _iota((Rc, k), 1)
    cx = c_ref[0][:, 0:1]
    cy = c_ref[0][:, 1:2]
    cz = c_ref[0][:, 2:3]
    px = p_ref[0, 0:1, :]
    py = p_ref[0, 1:2, :]
    pz = p_ref[0, 2:3, :]
    dx = cx - px
    dy = cy - py
    dz = cz - pz
    d2 = (dx * dx + dy * dy) + dz * dz
    ok = lane < n_pts
    if r2 is not None:
        ok = ok & (d2 <= r2)
    d2m_ref[...] = jnp.where(ok, d2, INF)

    nbr0 = jnp.zeros((Rc, k), I32)
    vld0 = jnp.zeros((Rc, k), I32)
    val0 = jnp.zeros((Rc, k), F32)

    def round_fn(r, nbr, vld, vals):
        d2m = d2m_ref[...]
        m = jnp.min(d2m, axis=1, keepdims=True)
        sel = jnp.min(jnp.where(d2m == m, lane, IBIG), axis=1, keepdims=True)
        has = m < INF
        upd = (col == r) & has
        nbr = jnp.where(upd, sel, nbr)
        vld = jnp.where(upd, 1, vld)
        vals = jnp.where(upd, m, vals)
        d2m_ref[...] = jnp.where(lane == sel, INF, d2m)
        return nbr, vld, vals, jnp.min(m) < INF

    if k > 4:
        def wbody(carry):
            r, nbr, vld, vals, _ = carry
            nbr, vld, vals, cont = round_fn(r, nbr, vld, vals)
            return r + 1, nbr, vld, vals, cont

        _, nbr, vld, vals, _ = lax.while_loop(
            lambda c: (c[0] < k) & c[4], wbody,
            (jnp.int32(0), nbr0, vld0, val0, True))
    else:
        nbr, vld, vals = nbr0, vld0, val0
        for r in range(k):
            nbr, vld, vals, _ = round_fn(jnp.int32(r), nbr, vld, vals)

    nbr_ref[0] = nbr
    if weighted:
        w = 1.0 / jnp.maximum(vals, 1e-16)
        aux_ref[0] = w / jnp.sum(w, axis=-1, keepdims=True)
    else:
        aux_ref[0] = vld


def _select(centers, pointsT, n_pts, k, r2, weighted, Rc=128):
    B, n_pad, _ = centers.shape
    Np = pointsT.shape[2]
    aux_dt = F32 if weighted else I32
    return pl.pallas_call(
        functools.partial(_select_body, k, n_pts, r2, weighted, Np, Rc),
        grid=(B, n_pad // Rc),
        in_specs=[
            pl.BlockSpec((1, Rc, 128), lambda b, i: (b, i, 0)),
            pl.BlockSpec((1, 8, Np), lambda b, i: (b, 0, 0)),
        ],
        out_specs=[
            pl.BlockSpec((1, Rc, k), lambda b, i: (b, i, 0)),
            pl.BlockSpec((1, Rc, k), lambda b, i: (b, i, 0)),
        ],
        out_shape=[
            jax.ShapeDtypeStruct((B, n_pad, k), I32),
            jax.ShapeDtypeStruct((B, n_pad, k), aux_dt),
        ],
        scratch_shapes=[pltpu.VMEM((Rc, Np), F32)],
        compiler_params=pltpu.CompilerParams(
            dimension_semantics=("parallel", "parallel")),
        interpret=False,
    )(centers, pointsT)


def _mm_body(has_bias, refs):
    if has_bias:
        x_ref, w_ref, b_ref, o_ref = refs
    else:
        x_ref, w_ref, o_ref = refs
    o = jnp.dot(x_ref[0], w_ref[...], preferred_element_type=F32)
    if has_bias:
        o = o + b_ref[...]
    o_ref[0] = o


def _mm(x, W, b=None, BM=512):
    B, M, K = x.shape
    H = W.shape[1]
    has_bias = b is not None
    ins = [x, W] + ([b.reshape(1, H)] if has_bias else [])
    in_specs = [
        pl.BlockSpec((1, BM, K), lambda bb, i: (bb, i, 0)),
        pl.BlockSpec((K, H), lambda bb, i: (0, 0)),
    ] + ([pl.BlockSpec((1, H), lambda bb, i: (0, 0))] if has_bias else [])
    return pl.pallas_call(
        lambda *refs: _mm_body(has_bias, refs),
        grid=(B, M // BM),
        in_specs=in_specs,
        out_specs=pl.BlockSpec((1, BM, H), lambda bb, i: (bb, i, 0)),
        out_shape=jax.ShapeDtypeStruct((B, M, H), F32),
        compiler_params=pltpu.CompilerParams(
            dimension_semantics=("parallel", "parallel")),
        interpret=False,
    )(*ins)


def _pair_mlp_body(Rc, H1, n_layers, refs):
    a_ref, nbr_ref, c_ref, wp_ref, vld_ref = refs[:5]
    wbs = refs[5:5 + 2 * n_layers]
    o_ref = refs[5 + 2 * n_layers]
    g_ref = refs[-1]

    def cprow(r, carry):
        for k in range(32):
            j = nbr_ref[0, r, k]
            g_ref[pl.ds(r * 32 + k, 1), :] = a_ref[0, pl.ds(j, 1), :]
        return carry

    lax.fori_loop(0, Rc, cprow, jnp.int32(0))

    cx = c_ref[0][:, 0:1]
    cy = c_ref[0][:, 1:2]
    cz = c_ref[0][:, 2:3]
    T = cx * wp_ref[0:1, :] + cy * wp_ref[1:2, :] + cz * wp_ref[2:3, :]
    g3 = g_ref[...].reshape(Rc, 32, H1)
    h3 = jnp.maximum(g3 - T[:, None, :], 0.0)
    h = h3.reshape(Rc * 32, H1)
    for li in range(n_layers):
        W = wbs[2 * li][...]
        b = wbs[2 * li + 1][...]
        h = jnp.maximum(jnp.dot(h, W, preferred_element_type=F32) + b, 0.0)
    HL = h.shape[1]
    h3 = h.reshape(Rc, 32, HL)
    vld3 = vld_ref[0][:, :, None] > 0
    o_ref[0] = jnp.max(jnp.where(vld3, h3, NEG), axis=1)


def _pair_mlp(A, nbr, centers, Wp, vld, layers, Rc=128):
    B, n_pad, _ = centers.shape
    np_prev = A.shape[1]
    H1 = A.shape[2]
    HL = layers[-1][0].shape[1] if layers else H1
    ins = [A, nbr, centers, Wp, vld]
    in_specs = [
        pl.BlockSpec((1, np_prev, H1), lambda b, i: (b, 0, 0)),
        pl.BlockSpec((1, Rc, 32), lambda b, i: (b, i, 0),
                     memory_space=pltpu.SMEM),
        pl.BlockSpec((1, Rc, 128), lambda b, i: (b, i, 0)),
        pl.BlockSpec(Wp.shape, lambda b, i: (0, 0)),
        pl.BlockSpec((1, Rc, 32), lambda b, i: (b, i, 0)),
    ]
    for (W, bb) in layers:
        ins += [W, bb.reshape(1, -1)]
        in_specs += [pl.BlockSpec(W.shape, lambda b, i: (0, 0)),
                     pl.BlockSpec((1, bb.shape[0]), lambda b, i: (0, 0))]
    return pl.pallas_call(
        lambda *refs: _pair_mlp_body(Rc, H1, len(layers), refs),
        grid=(B, n_pad // Rc),
        in_specs=in_specs,
        out_specs=pl.BlockSpec((1, Rc, HL), lambda b, i: (b, i, 0)),
        out_shape=jax.ShapeDtypeStruct((B, n_pad, HL), F32),
        scratch_shapes=[pltpu.VMEM((Rc * 32, H1), F32)],
        compiler_params=pltpu.CompilerParams(
            dimension_semantics=("parallel", "parallel")),
        interpret=False,
    )(*ins)


def _fp_mlp(Bmat, nbr, wts, skip, pos_rows, w1b, w1c, b1, layers,
            resid=None, Rc=128):
    B, n_pad, _ = nbr.shape
    np_c = Bmat.shape[1]
    H1 = Bmat.shape[2]
    HL = layers[-1][0].shape[1] if layers else H1

    def body(*refs):
        bm_ref, nbr_ref, w_ref, sk_ref, pr_ref, w1b_r, w1c_r, b1_r = refs[:8]
        p = 8
        h_w = []
        for (W, bb, do_relu) in layers:
            Wr = refs[p]
            p += 1
            br = None
            if bb is not None:
                br = refs[p]
                p += 1
            h_w.append((Wr, br, do_relu))
        resid_ref = refs[p] if resid is not None else None
        if resid is not None:
            p += 1
        o_ref = refs[p]
        g_ref = refs[-1]

        def cprow(r, carry):
            for k in range(3):
                j = nbr_ref[0, r, k]
                g_ref[pl.ds(k * Rc + r, 1), :] = bm_ref[0, pl.ds(j, 1), :]
            return carry

        lax.fori_loop(0, Rc, cprow, jnp.int32(0))

        w = w_ref[0]
        g = g_ref[...]
        interp = ((w[:, 0:1] * g[:Rc] + w[:, 1:2] * g[Rc:2 * Rc])
                  + w[:, 2:3] * g[2 * Rc:])
        sk = jnp.dot(sk_ref[0], w1b_r[...], preferred_element_type=F32)
        px = pr_ref[0][:, 0:1]
        py = pr_ref[0][:, 1:2]
        pz = pr_ref[0][:, 2:3]
        pp = px * w1c_r[0:1, :] + py * w1c_r[1:2, :] + pz * w1c_r[2:3, :]
        h = jnp.maximum(interp + sk + pp + b1_r[...], 0.0)
        for (Wr, br, do_relu) in h_w:
            h = jnp.dot(h, Wr[...], preferred_element_type=F32)
            if br is not None:
                h = h + br[...]
            if do_relu:
                h = jnp.maximum(h, 0.0)
        if resid_ref is not None:
            h = h + resid_ref[0]
        o_ref[0] = h

    Cs = skip.shape[2]
    ins = [Bmat, nbr, wts, skip, pos_rows, w1b, w1c, b1.reshape(1, -1)]
    in_specs = [
        pl.BlockSpec((1, np_c, H1), lambda b, i: (b, 0, 0)),
        pl.BlockSpec((1, Rc, 3), lambda b, i: (b, i, 0),
                     memory_space=pltpu.SMEM),
        pl.BlockSpec((1, Rc, 3), lambda b, i: (b, i, 0)),
        pl.BlockSpec((1, Rc, Cs), lambda b, i: (b, i, 0)),
        pl.BlockSpec((1, Rc, 128), lambda b, i: (b, i, 0)),
        pl.BlockSpec(w1b.shape, lambda b, i: (0, 0)),
        pl.BlockSpec(w1c.shape, lambda b, i: (0, 0)),
        pl.BlockSpec((1, b1.shape[0]), lambda b, i: (0, 0)),
    ]
    for (W, bb, _r) in layers:
        ins.append(W)
        in_specs.append(pl.BlockSpec(W.shape, lambda b, i: (0, 0)))
        if bb is not None:
            ins.append(bb.reshape(1, -1))
            in_specs.append(pl.BlockSpec((1, bb.shape[0]),
                                         lambda b, i: (0, 0)))
    if resid is not None:
        ins.append(resid)
        in_specs.append(pl.BlockSpec((1, Rc, resid.shape[2]),
                                     lambda b, i: (b, i, 0)))
    return pl.pallas_call(
        body,
        grid=(B, n_pad // Rc),
        in_specs=in_specs,
        out_specs=pl.BlockSpec((1, Rc, HL), lambda b, i: (b, i, 0)),
        out_shape=jax.ShapeDtypeStruct((B, n_pad, HL), F32),
        scratch_shapes=[pltpu.VMEM((3 * Rc, H1), F32)],
        compiler_params=pltpu.CompilerParams(
            dimension_semantics=("parallel", "parallel")),
        interpret=False,
    )(*ins)


def _rows_to_planes(rows, n_pad):
    t = rows[:, :, :3].transpose(0, 2, 1)
    return t.reshape(2, 3, 8, n_pad // 8)


def _rows_to_pT(rows):
    t = rows[:, :, :3].transpose(0, 2, 1)
    return jnp.pad(t, ((0, 0), (0, 5), (0, 0)))


def _sa_stage(prm, x_prev, pos_rows, planes, pT, n_prev, np_prev,
              n_s, np_s, radius):
    W1, b1 = prm[0]
    Cx = x_prev.shape[2]
    centers = _fps(planes, n_s, np_s, n_prev)
    nbr, vld = _select(centers, pT, n_prev, 32, radius * radius, False)
    xp = jnp.concatenate([x_prev, pos_rows[:, :, :3]], axis=-1)
    A = _mm(xp, W1, b1)
    x_s = _pair_mlp(A, nbr, centers, W1[Cx:], vld, prm[1:])
    return x_s, centers


def _fp_stage(prm, x_c, pos_c_rows, x_skip, pos_f_rows, n_c, np_c,
              n_f, np_f, extra_fc=None, resid=None):
    W1, b1 = prm[0]
    Cc = x_c.shape[2]
    Cs = x_skip.shape[2]
    pT_c = _rows_to_pT(pos_c_rows)
    nbr, w = _select(pos_f_rows, pT_c, n_c, 3, None, True)
    Bmat = _mm(x_c, W1[:Cc])
    layers = [(W, b, True) for (W, b) in prm[1:]]
    if extra_fc is not None:
        layers.append((extra_fc[0], None, True))
        for Wf in extra_fc[1:]:
            layers.append((Wf, None, False))
    return _fp_mlp(Bmat, nbr, w, x_skip, pos_f_rows,
                   W1[Cc:Cc + Cs], W1[Cc + Cs:], b1, layers, resid=resid)


def kernel(data, params):
    n = [4096, 2868, 2008, 1406, 985]
    npad = [4096, 3072, 2048, 1536, 1024]
    radii = [0.025, 0.05, 0.1, 0.2]

    pos0_rows3 = data.transpose(0, 2, 1)
    pos0_rows = jnp.pad(pos0_rows3, ((0, 0), (0, 0), (0, 125)))
    planes0 = data.reshape(2, 3, 8, 512)
    pT0 = jnp.pad(data, ((0, 0), (0, 5), (0, 0)))

    xs = [pos0_rows3]
    ps = [pos0_rows]
    planes, pT = planes0, pT0
    for s in range(4):
        prm = params['sa' + str(s + 1)]
        x_s, centers = _sa_stage(
            prm, xs[s], ps[s], planes, pT,
            n[s], npad[s], n[s + 1], npad[s + 1], radii[s])
        xs.append(x_s)
        ps.append(centers)
        if s < 3:
            planes = _rows_to_planes(centers, npad[s + 1])
            pT = _rows_to_pT(centers)

    f4 = _fp_stage(params['fp4'], xs[4], ps[4], xs[3], ps[3],
                   n[4], npad[4], n[3], npad[3])
    f3 = _fp_stage(params['fp3'], f4, ps[3], xs[2], ps[2],
                   n[3], npad[3], n[2], npad[2])
    f2 = _fp_stage(params['fp2'], f3, ps[2], xs[1], ps[1],
                   n[2], npad[2], n[1], npad[1])
    out = _fp_stage(params['fp1'], f2, ps[1], pos0_rows3, ps[0],
                    n[1], npad[1], n[0], npad[0],
                    extra_fc=params['fc'], resid=pos0_rows3)
    return out.transpose(0, 2, 1)
